# baseline (speedup 1.0000x reference)
import functools

import jax
import jax.numpy as jnp
from jax import lax
from jax.experimental import pallas as pl
from jax.experimental.pallas import tpu as pltpu
from jax.experimental.pallas import tpu_sc as plsc

D_MODEL = 1024
N_TOKENS = 4 * 4096
NUM_CORES = 2
NUM_SUBCORES = 16
NUM_WORKERS = NUM_CORES * NUM_SUBCORES
TOK_PER_WORKER = N_TOKENS // NUM_WORKERS
CHUNK = 32
NUM_CHUNKS = TOK_PER_WORKER // CHUNK
NBUF = 2
LANES = 16

_mesh = plsc.VectorSubcoreMesh(core_axis_name="c", subcore_axis_name="s")


def _splat(vec16, t):
    starts = jnp.full((LANES,), t, dtype=jnp.int32).reshape(LANES, 1)
    return lax.gather(
        vec16,
        starts,
        lax.GatherDimensionNumbers(
            offset_dims=(), collapsed_slice_dims=(0,), start_index_map=(0,)
        ),
        (1,),
        mode=lax.GatherScatterMode.PROMISE_IN_BOUNDS,
    )


@functools.partial(
    pl.kernel,
    mesh=_mesh,
    out_type=jax.ShapeDtypeStruct((N_TOKENS, D_MODEL), jnp.float32),
    scratch_types=[
        pltpu.VMEM((TOK_PER_WORKER,), jnp.int32),
        pltpu.VMEM((2, D_MODEL), jnp.float32),
        pltpu.VMEM((CHUNK * LANES,), jnp.int32),
        pltpu.VMEM((NBUF * CHUNK, D_MODEL), jnp.float32),
        pltpu.SemaphoreType.DMA,
    ],
)
def _embed(idx_hbm, table_hbm, out_hbm, idx_v, table_v, m_v, ring, sem):
    wid = lax.axis_index("s") * NUM_CORES + lax.axis_index("c")
    base = wid * TOK_PER_WORKER
    pltpu.sync_copy(idx_hbm.at[pl.ds(base, TOK_PER_WORKER)], idx_v)
    pltpu.sync_copy(table_hbm, table_v)

    def body(g, carry):
        buf = ring.at[pl.ds(lax.rem(g, NBUF) * CHUNK, CHUNK)]

        @pl.when(g >= NBUF)
        def _wait_prev():
            pltpu.make_async_copy(buf, out_hbm.at[pl.ds(0, CHUNK)], sem).wait()

        for h in range(CHUNK // LANES):
            ids16 = idx_v[pl.ds(g * CHUNK + h * LANES, LANES)]
            for t in range(LANES):
                m_v[pl.ds((h * LANES + t) * LANES, LANES)] = _splat(ids16, t)

        @plsc.parallel_loop(0, D_MODEL // LANES, unroll=8)
        def jbody(j):
            w0 = table_v[0, pl.ds(j * LANES, LANES)]
            w1 = table_v[1, pl.ds(j * LANES, LANES)]
            for t in range(CHUNK):
                sel = m_v[pl.ds(t * LANES, LANES)] != 0
                buf[t, pl.ds(j * LANES, LANES)] = jnp.where(sel, w1, w0)

        pltpu.async_copy(buf, out_hbm.at[pl.ds(base + g * CHUNK, CHUNK)], sem)
        return carry

    lax.fori_loop(0, NUM_CHUNKS, body, 0)
    for _ in range(NBUF):
        pltpu.make_async_copy(
            ring.at[pl.ds(0, CHUNK)], out_hbm.at[pl.ds(0, CHUNK)], sem
        ).wait()


def kernel(token_type_ids, W_token_type):
    out = _embed(token_type_ids.reshape(N_TOKENS), W_token_type)
    return out.reshape(token_type_ids.shape[0], token_type_ids.shape[1], D_MODEL)


# speedup vs baseline: 1.1119x; 1.1119x over previous
import functools

import jax
import jax.numpy as jnp
from jax import lax
from jax.experimental import pallas as pl
from jax.experimental.pallas import tpu as pltpu
from jax.experimental.pallas import tpu_sc as plsc

D_MODEL = 1024
N_TOKENS = 4 * 4096
NUM_CORES = 2
NUM_SUBCORES = 16
NUM_WORKERS = NUM_CORES * NUM_SUBCORES
TOK_PER_WORKER = N_TOKENS // NUM_WORKERS
CHUNK = 16
NUM_CHUNKS = TOK_PER_WORKER // CHUNK
NBUF = 2
LANES = 16

_mesh = plsc.VectorSubcoreMesh(core_axis_name="c", subcore_axis_name="s")


def _splat(vec16, t):
    starts = jnp.full((LANES,), t, dtype=jnp.int32).reshape(LANES, 1)
    return lax.gather(
        vec16,
        starts,
        lax.GatherDimensionNumbers(
            offset_dims=(), collapsed_slice_dims=(0,), start_index_map=(0,)
        ),
        (1,),
        mode=lax.GatherScatterMode.PROMISE_IN_BOUNDS,
    )


@functools.partial(
    pl.kernel,
    mesh=_mesh,
    out_type=jax.ShapeDtypeStruct((N_TOKENS, D_MODEL), jnp.float32),
    scratch_types=[
        pltpu.VMEM((TOK_PER_WORKER,), jnp.int32),
        pltpu.VMEM((2, D_MODEL), jnp.float32),
        pltpu.VMEM((CHUNK * LANES,), jnp.int32),
        pltpu.VMEM((NBUF * CHUNK, D_MODEL), jnp.float32),
        pltpu.SemaphoreType.DMA,
    ],
)
def _embed(idx_hbm, table_hbm, out_hbm, idx_v, table_v, m_v, ring, sem):
    wid = lax.axis_index("s") * NUM_CORES + lax.axis_index("c")
    base = wid * TOK_PER_WORKER
    pltpu.sync_copy(idx_hbm.at[pl.ds(base, TOK_PER_WORKER)], idx_v)
    pltpu.sync_copy(table_hbm, table_v)

    def body(g, carry):
        buf = ring.at[pl.ds(lax.rem(g, NBUF) * CHUNK, CHUNK)]

        @pl.when(g >= NBUF)
        def _wait_prev():
            pltpu.make_async_copy(buf, out_hbm.at[pl.ds(0, CHUNK)], sem).wait()

        for h in range(CHUNK // LANES):
            ids16 = idx_v[pl.ds(g * CHUNK + h * LANES, LANES)]
            for t in range(LANES):
                m_v[pl.ds((h * LANES + t) * LANES, LANES)] = _splat(ids16, t)

        @plsc.parallel_loop(0, D_MODEL // LANES, unroll=4)
        def jbody(j):
            w0 = table_v[0, pl.ds(j * LANES, LANES)]
            w1 = table_v[1, pl.ds(j * LANES, LANES)]
            for t in range(CHUNK):
                sel = m_v[pl.ds(t * LANES, LANES)] != 0
                buf[t, pl.ds(j * LANES, LANES)] = jnp.where(sel, w1, w0)

        pltpu.async_copy(buf, out_hbm.at[pl.ds(base + g * CHUNK, CHUNK)], sem)
        return carry

    lax.fori_loop(0, NUM_CHUNKS, body, 0)
    for _ in range(NBUF):
        pltpu.make_async_copy(
            ring.at[pl.ds(0, CHUNK)], out_hbm.at[pl.ds(0, CHUNK)], sem
        ).wait()


def kernel(token_type_ids, W_token_type):
    out = _embed(token_type_ids.reshape(N_TOKENS), W_token_type)
    return out.reshape(token_type_ids.shape[0], token_type_ids.shape[1], D_MODEL)
